# baseline (speedup 1.0000x reference)
import functools

import jax
import jax.numpy as jnp
from jax import lax
from jax.experimental import pallas as pl
from jax.experimental.pallas import tpu as pltpu
from jax.experimental.pallas import tpu_sc as plsc

_B = 16384
_D = 128
_NC = 2
_NS = 16
_NW = _NC * _NS
_BPW = _B // _NW
_CH = 128
_NCHUNK = _BPW // _CH

_mesh = plsc.VectorSubcoreMesh(core_axis_name="c", subcore_axis_name="s")


@functools.partial(
    pl.kernel,
    mesh=_mesh,
    out_type=jax.ShapeDtypeStruct((_B,), jnp.float32),
    scratch_types=[
        pltpu.VMEM((_CH,), jnp.int32),
        pltpu.VMEM((_CH,), jnp.int32),
        pltpu.VMEM((_CH,), jnp.int32),
        pltpu.VMEM((_CH,), jnp.int32),
        pltpu.VMEM((_CH,), jnp.int32),
        pltpu.VMEM((_CH,), jnp.int32),
        pltpu.VMEM((_CH, _D), jnp.float32),
        pltpu.VMEM((_CH, _D), jnp.float32),
        pltpu.VMEM((_CH, _D), jnp.float32),
        pltpu.VMEM((_CH, _D), jnp.float32),
        pltpu.VMEM((_CH, _D), jnp.float32),
        pltpu.VMEM((_CH, _D), jnp.float32),
        pltpu.VMEM((_CH,), jnp.float32),
        pltpu.VMEM((_CH,), jnp.float32),
        pltpu.VMEM((_CH,), jnp.float32),
        pltpu.VMEM((_CH,), jnp.float32),
        pltpu.VMEM((_CH,), jnp.float32),
        pltpu.VMEM((512,), jnp.float32),
        pltpu.SemaphoreType.DMA,
        pltpu.SemaphoreType.DMA,
    ],
)
def _cplr_sc(users, pos_items, neg_items, item_biases, ue, ie, out,
             iu0, ip0, in0, iu1, ip1, in1,
             ru0, rp0, rn0, ru1, rp1, rn1,
             pb0, nb0, pb1, nb1, out_v, tmp, sem0, sem1):
    wid = lax.axis_index("s") * _NC + lax.axis_index("c")
    base = wid * _BPW
    lanes = lax.iota(jnp.int32, 16)

    idx_sets = [(iu0, ip0, in0), (iu1, ip1, in1)]
    row_sets = [(ru0, rp0, rn0), (ru1, rp1, rn1)]
    bias_sets = [(pb0, nb0), (pb1, nb1)]
    sems = [sem0, sem1]

    def stage_and_fire(c):
        s = c % 2
        iu, ip_, in_ = idx_sets[s]
        ru, rp, rn = row_sets[s]
        pb, nb = bias_sets[s]
        off = base + c * _CH
        pltpu.sync_copy(users.at[pl.ds(off, _CH)], iu)
        pltpu.sync_copy(pos_items.at[pl.ds(off, _CH)], ip_)
        pltpu.sync_copy(neg_items.at[pl.ds(off, _CH)], in_)
        return [
            pltpu.async_copy(ue.at[iu], ru, sems[s]),
            pltpu.async_copy(ie.at[ip_], rp, sems[s]),
            pltpu.async_copy(ie.at[in_], rn, sems[s]),
            pltpu.async_copy(item_biases.at[ip_], pb, sems[s]),
            pltpu.async_copy(item_biases.at[in_], nb, sems[s]),
        ]

    pending = {0: stage_and_fire(0)}
    for c in range(_NCHUNK):
        s = c % 2
        if c + 1 < _NCHUNK:
            pending[c + 1] = stage_and_fire(c + 1)
        for cp in pending.pop(c):
            cp.wait()
        ru, rp, rn = row_sets[s]
        pb, nb = bias_sets[s]
        off = base + c * _CH

        def group_body(g, carry, ru=ru, rp=rp, rn=rn, pb=pb, nb=nb):
            e0 = g * 16
            tot = jnp.zeros((16,), jnp.float32)
            for i in range(16):
                e = e0 + i
                prods = []
                for j in range(_D // 16):
                    u = ru[e, pl.ds(j * 16, 16)]
                    p = rp[e, pl.ds(j * 16, 16)]
                    n = rn[e, pl.ds(j * 16, 16)]
                    prods.append(u * (p - n))
                while len(prods) > 1:
                    prods = [prods[k] + prods[k + 1]
                             for k in range(0, len(prods), 2)]
                acc = prods[0]
                b = 32 * i
                tmp[pl.ds(b, 16)] = acc
                s1 = acc + tmp[pl.ds(b + 8, 16)]
                tmp[pl.ds(b, 16)] = s1
                s2 = s1 + tmp[pl.ds(b + 4, 16)]
                tmp[pl.ds(b, 16)] = s2
                s3 = s2 + tmp[pl.ds(b + 2, 16)]
                tmp[pl.ds(b, 16)] = s3
                s4 = s3 + tmp[pl.ds(b + 1, 16)]
                tot = jnp.where(lanes == i, s4[0], tot)
            out_v[pl.ds(e0, 16)] = (tot + pb[pl.ds(e0, 16)]
                                    - nb[pl.ds(e0, 16)])
            return carry

        lax.fori_loop(0, _CH // 16, group_body, 0)
        pltpu.sync_copy(out_v, out.at[pl.ds(off, _CH)])


def kernel(users, pos_items, neg_items, user_biases, item_biases,
           user_embeddings, item_embeddings):
    del user_biases
    return _cplr_sc(
        users.astype(jnp.int32),
        pos_items.astype(jnp.int32),
        neg_items.astype(jnp.int32),
        item_biases.reshape(-1),
        user_embeddings,
        item_embeddings,
    )


# speedup vs baseline: 1.1932x; 1.1932x over previous
import functools

import jax
import jax.numpy as jnp
from jax import lax
from jax.experimental import pallas as pl
from jax.experimental.pallas import tpu as pltpu
from jax.experimental.pallas import tpu_sc as plsc

_B = 16384
_D = 128
_NC = 2
_NS = 16
_NW = _NC * _NS
_BPW = _B // _NW
_CH = 128
_NCHUNK = _BPW // _CH

_mesh = plsc.VectorSubcoreMesh(core_axis_name="c", subcore_axis_name="s")


@functools.partial(
    pl.kernel,
    mesh=_mesh,
    out_type=jax.ShapeDtypeStruct((_B,), jnp.float32),
    scratch_types=[
        pltpu.VMEM((_CH,), jnp.int32),
        pltpu.VMEM((_CH,), jnp.int32),
        pltpu.VMEM((_CH,), jnp.int32),
        pltpu.VMEM((_CH,), jnp.int32),
        pltpu.VMEM((_CH,), jnp.int32),
        pltpu.VMEM((_CH,), jnp.int32),
        pltpu.VMEM((_CH, _D), jnp.float32),
        pltpu.VMEM((_CH, _D), jnp.float32),
        pltpu.VMEM((_CH, _D), jnp.float32),
        pltpu.VMEM((_CH, _D), jnp.float32),
        pltpu.VMEM((_CH, _D), jnp.float32),
        pltpu.VMEM((_CH, _D), jnp.float32),
        pltpu.VMEM((_CH,), jnp.float32),
        pltpu.VMEM((_CH,), jnp.float32),
        pltpu.VMEM((_CH,), jnp.float32),
        pltpu.VMEM((_CH,), jnp.float32),
        pltpu.VMEM((_CH,), jnp.float32),
        pltpu.VMEM((512,), jnp.float32),
        pltpu.SemaphoreType.DMA,
        pltpu.SemaphoreType.DMA,
    ],
)
def _cplr_sc(users, pos_items, neg_items, item_biases, ue, ie, out,
             iu0, ip0, in0, iu1, ip1, in1,
             ru0, rp0, rn0, ru1, rp1, rn1,
             pb0, nb0, pb1, nb1, out_v, tmp, sem0, sem1):
    wid = lax.axis_index("s") * _NC + lax.axis_index("c")
    base = wid * _BPW
    lanes = lax.iota(jnp.int32, 16)

    idx_sets = [(iu0, ip0, in0), (iu1, ip1, in1)]
    row_sets = [(ru0, rp0, rn0), (ru1, rp1, rn1)]
    bias_sets = [(pb0, nb0), (pb1, nb1)]
    sems = [sem0, sem1]

    def stage_and_fire(c):
        s = c % 2
        iu, ip_, in_ = idx_sets[s]
        ru, rp, rn = row_sets[s]
        pb, nb = bias_sets[s]
        off = base + c * _CH
        pltpu.sync_copy(users.at[pl.ds(off, _CH)], iu)
        pltpu.sync_copy(pos_items.at[pl.ds(off, _CH)], ip_)
        pltpu.sync_copy(neg_items.at[pl.ds(off, _CH)], in_)
        return [
            pltpu.async_copy(ue.at[iu], ru, sems[s]),
            pltpu.async_copy(ie.at[ip_], rp, sems[s]),
            pltpu.async_copy(ie.at[in_], rn, sems[s]),
            pltpu.async_copy(item_biases.at[ip_], pb, sems[s]),
            pltpu.async_copy(item_biases.at[in_], nb, sems[s]),
        ]

    pending = {0: stage_and_fire(0)}
    for c in range(_NCHUNK):
        s = c % 2
        if c + 1 < _NCHUNK:
            pending[c + 1] = stage_and_fire(c + 1)
        for cp in pending.pop(c):
            cp.wait()
        ru, rp, rn = row_sets[s]
        pb, nb = bias_sets[s]
        off = base + c * _CH

        def group_body(g, carry, ru=ru, rp=rp, rn=rn, pb=pb, nb=nb):
            e0 = g * 16
            tot = jnp.zeros((16,), jnp.float32)
            for i in range(16):
                e = e0 + i
                prods = []
                for j in range(_D // 16):
                    u = ru[e, pl.ds(j * 16, 16)]
                    p = rp[e, pl.ds(j * 16, 16)]
                    n = rn[e, pl.ds(j * 16, 16)]
                    prods.append(u * (p - n))
                while len(prods) > 1:
                    prods = [prods[k] + prods[k + 1]
                             for k in range(0, len(prods), 2)]
                acc = prods[0]
                parts = [acc[k] for k in range(16)]
                while len(parts) > 1:
                    parts = [parts[k] + parts[k + 1]
                             for k in range(0, len(parts), 2)]
                tot = jnp.where(lanes == i, parts[0], tot)
            out_v[pl.ds(e0, 16)] = (tot + pb[pl.ds(e0, 16)]
                                    - nb[pl.ds(e0, 16)])
            return carry

        lax.fori_loop(0, _CH // 16, group_body, 0)
        pltpu.sync_copy(out_v, out.at[pl.ds(off, _CH)])


def kernel(users, pos_items, neg_items, user_biases, item_biases,
           user_embeddings, item_embeddings):
    del user_biases
    return _cplr_sc(
        users.astype(jnp.int32),
        pos_items.astype(jnp.int32),
        neg_items.astype(jnp.int32),
        item_biases.reshape(-1),
        user_embeddings,
        item_embeddings,
    )
